# Initial kernel scaffold; baseline (speedup 1.0000x reference)
import jax
import jax.numpy as jnp
from jax.experimental import pallas as pl


def kernel(x, embeddings):
    raise NotImplementedError("write your pallas kernel here")


# speedup vs baseline: 1.9042x; 1.9042x over previous
import functools

import jax
import jax.numpy as jnp
from jax.experimental import pallas as pl


def _vq_tile_kernel(x_ref, e_ref, out_ref, *, num_codes):
    e = e_ref[...]
    xt = x_ref[0]
    scores = jax.lax.dot_general(
        e, xt, (((1,), (0,)), ((), ())),
        preferred_element_type=jnp.float32)
    e_norm = jnp.sum(e * e, axis=1, keepdims=True)
    dist = e_norm - 2.0 * scores
    m = jnp.min(dist, axis=0, keepdims=True)
    k_iota = jax.lax.broadcasted_iota(jnp.int32, dist.shape, 0)
    idx = jnp.min(jnp.where(dist == m, k_iota, num_codes),
                  axis=0, keepdims=True)
    one_hot = (k_iota == idx).astype(jnp.float32)
    q = jax.lax.dot_general(
        e, one_hot, (((0,), (0,)), ((), ())),
        preferred_element_type=jnp.float32)
    out_ref[0] = q


@functools.partial(jax.jit, static_argnames=("interpret",))
def kernel(x, embeddings, interpret=False):
    B, D, T = x.shape
    K = embeddings.shape[0]
    TT = 256
    quantized = pl.pallas_call(
        functools.partial(_vq_tile_kernel, num_codes=K),
        grid=(B, T // TT),
        in_specs=[
            pl.BlockSpec((1, D, TT), lambda b, t: (b, 0, t)),
            pl.BlockSpec((K, D), lambda b, t: (0, 0)),
        ],
        out_specs=pl.BlockSpec((1, D, TT), lambda b, t: (b, 0, t)),
        out_shape=jax.ShapeDtypeStruct((B, D, T), jnp.float32),
        interpret=interpret,
    )(x, embeddings)
    return (quantized, x, quantized)
